# Initial kernel scaffold; baseline (speedup 1.0000x reference)
import functools

import jax
import jax.numpy as jnp
from jax import lax
from jax.experimental import pallas as pl
from jax.experimental.pallas import tpu as pltpu
from jax.experimental.pallas import tpu_sc as plsc

N = 50000
E = 800000
NFEAT = 128
NHID = 128
NCLASS = 64
K = 8

NC = 2
NS = 16
LANES = 16

ROWS_PER_TILE = 1600
NPAD = NC * NS * ROWS_PER_TILE
HALF = NS * ROWS_PER_TILE

EC = 128
CHUNK_ROWS = 8
CHUNK_E = CHUNK_ROWS * EC
EDGES_PER_TILE = 51200
EPAD = NS * EDGES_PER_TILE
NCHUNKS = EDGES_PER_TILE // CHUNK_E
EROWS = EPAD // EC
ZROWS = 200


def _mlp_body(x_ref, w1_ref, b1_ref, w2_ref, b2_ref, o_ref):
    h = jnp.dot(x_ref[...], w1_ref[...], preferred_element_type=jnp.float32)
    h = jnp.maximum(h + b1_ref[...], 0.0)
    o = jnp.dot(h, w2_ref[...], preferred_element_type=jnp.float32)
    o_ref[...] = o + b2_ref[...]


_MLP_BM = 2048
_MLP_GRID = NPAD // _MLP_BM


def _mlp(xp, W1, b1, W2, b2):
    return pl.pallas_call(
        _mlp_body,
        grid=(_MLP_GRID,),
        in_specs=[
            pl.BlockSpec((_MLP_BM, NFEAT), lambda i: (i, 0)),
            pl.BlockSpec((NFEAT, NHID), lambda i: (0, 0)),
            pl.BlockSpec((1, NHID), lambda i: (0, 0)),
            pl.BlockSpec((NHID, NCLASS), lambda i: (0, 0)),
            pl.BlockSpec((1, NCLASS), lambda i: (0, 0)),
        ],
        out_specs=pl.BlockSpec((_MLP_BM, NCLASS), lambda i: (i, 0)),
        out_shape=jax.ShapeDtypeStruct((NPAD, NCLASS), jnp.float32),
    )(xp, W1, b1.reshape(1, NHID), W2, b2.reshape(1, NCLASS))


def _hop_body(src_hbm, dst_hbm, val_hbm, in_hbm, out_hbm,
              acc_sh, src_v, dst_v, val_v, rows_v, zbuf, sem):
    core = lax.axis_index("c")
    sid = lax.axis_index("s")
    base = core * HALF

    def zbody(r, _):
        for c in range(NCLASS // LANES):
            zbuf[r, pl.ds(c * LANES, LANES)] = jnp.zeros((LANES,), jnp.float32)
        return 0
    lax.fori_loop(0, ZROWS, zbody, 0)
    for q in range(ROWS_PER_TILE // ZROWS):
        pltpu.sync_copy(
            zbuf, acc_sh.at[pl.ds(sid * ROWS_PER_TILE + q * ZROWS, ZROWS)])
    plsc.subcore_barrier()

    def chunk_body(ci, _):
        row0 = sid * (EDGES_PER_TILE // EC) + ci * CHUNK_ROWS
        pltpu.sync_copy(src_hbm.at[pl.ds(row0, CHUNK_ROWS)], src_v)
        pltpu.sync_copy(dst_hbm.at[pl.ds(row0, CHUNK_ROWS)], dst_v)
        pltpu.sync_copy(val_hbm.at[pl.ds(row0, CHUNK_ROWS)], val_v)

        descs = [
            pltpu.async_copy(in_hbm.at[src_v.at[j]], rows_v.at[j], sem)
            for j in range(CHUNK_ROWS)
        ]
        for j in range(CHUNK_ROWS):
            for kk in range(EC // LANES):
                sl = pl.ds(kk * LANES, LANES)
                rel = dst_v[j, sl] - base
                inr = (rel >= 0) & (rel < HALF)
                dst_v[j, sl] = jnp.where(inr, rel, 0)
                val_v[j, sl] = jnp.where(inr, val_v[j, sl], 0.0)
        for d in descs:
            d.wait()

        for j in range(CHUNK_ROWS):
            def mbody(e, _, j=j):
                v = val_v[j, e]
                vb = jnp.full((LANES,), v, jnp.float32)
                for c in range(NCLASS // LANES):
                    sl = pl.ds(c * LANES, LANES)
                    rows_v[j, e, sl] = rows_v[j, e, sl] * vb
                return 0
            lax.fori_loop(0, EC, mbody, 0)

        for j in range(CHUNK_ROWS):
            pltpu.sync_copy(rows_v.at[j], acc_sh.at[dst_v.at[j]], add=True)
        return 0

    lax.fori_loop(0, NCHUNKS, chunk_body, 0)
    plsc.subcore_barrier()

    pltpu.sync_copy(
        acc_sh.at[pl.ds(sid * ROWS_PER_TILE, ROWS_PER_TILE)],
        out_hbm.at[pl.ds(base + sid * ROWS_PER_TILE, ROWS_PER_TILE)])


_hop = functools.partial(
    pl.kernel,
    out_type=jax.ShapeDtypeStruct((NPAD, NCLASS), jnp.float32),
    mesh=plsc.VectorSubcoreMesh(core_axis_name="c", subcore_axis_name="s"),
    scratch_types=[
        pltpu.VMEM_SHARED((HALF, NCLASS), jnp.float32),
        pltpu.VMEM((CHUNK_ROWS, EC), jnp.int32),
        pltpu.VMEM((CHUNK_ROWS, EC), jnp.int32),
        pltpu.VMEM((CHUNK_ROWS, EC), jnp.float32),
        pltpu.VMEM((CHUNK_ROWS, EC, NCLASS), jnp.float32),
        pltpu.VMEM((ZROWS, NCLASS), jnp.float32),
        pltpu.SemaphoreType.DMA,
    ],
)(_hop_body)


def kernel(x, adj_values, W1, b1, W2, b2, adj_indices):
    dst = adj_indices[0]
    src = adj_indices[1]
    epad = EPAD - E
    srcp = jnp.concatenate([src, jnp.zeros((epad,), jnp.int32)]).reshape(EROWS, EC)
    dstp = jnp.concatenate([dst, jnp.zeros((epad,), jnp.int32)]).reshape(EROWS, EC)
    valp = jnp.concatenate(
        [adj_values, jnp.zeros((epad,), jnp.float32)]).reshape(EROWS, EC)
    xp = jnp.pad(x, ((0, NPAD - N), (0, 0)))

    h = _mlp(xp, W1, b1, W2, b2)
    for _ in range(K):
        h = _hop(srcp, dstp, valp, h)
    return h[:N]


# speedup vs baseline: 1.6228x; 1.6228x over previous
import functools

import jax
import jax.numpy as jnp
from jax import lax
from jax.experimental import pallas as pl
from jax.experimental.pallas import tpu as pltpu
from jax.experimental.pallas import tpu_sc as plsc

N = 50000
E = 800000
NFEAT = 128
NHID = 128
NCLASS = 64
K = 8

NC = 2
NS = 16
LANES = 16

ROWS_PER_TILE = 1600
NPAD = NC * NS * ROWS_PER_TILE
HALF = NS * ROWS_PER_TILE

EC = 128
CHUNK_ROWS = 2
CHUNK_E = CHUNK_ROWS * EC
EDGES_PER_TILE = 51200
EPAD = NS * EDGES_PER_TILE
NCHUNKS = EDGES_PER_TILE // CHUNK_E
EROWS = EPAD // EC
ZROWS = 100


def _mlp_body(x_ref, w1_ref, b1_ref, w2_ref, b2_ref, o_ref):
    h = jnp.dot(x_ref[...], w1_ref[...], preferred_element_type=jnp.float32)
    h = jnp.maximum(h + b1_ref[...], 0.0)
    o = jnp.dot(h, w2_ref[...], preferred_element_type=jnp.float32)
    o_ref[...] = o + b2_ref[...]


_MLP_BM = 2048
_MLP_GRID = NPAD // _MLP_BM


def _mlp(xp, W1, b1, W2, b2):
    return pl.pallas_call(
        _mlp_body,
        grid=(_MLP_GRID,),
        in_specs=[
            pl.BlockSpec((_MLP_BM, NFEAT), lambda i: (i, 0)),
            pl.BlockSpec((NFEAT, NHID), lambda i: (0, 0)),
            pl.BlockSpec((1, NHID), lambda i: (0, 0)),
            pl.BlockSpec((NHID, NCLASS), lambda i: (0, 0)),
            pl.BlockSpec((1, NCLASS), lambda i: (0, 0)),
        ],
        out_specs=pl.BlockSpec((_MLP_BM, NCLASS), lambda i: (i, 0)),
        out_shape=jax.ShapeDtypeStruct((NPAD, NCLASS), jnp.float32),
    )(xp, W1, b1.reshape(1, NHID), W2, b2.reshape(1, NCLASS))


def _hop_body(src_hbm, dst_hbm, val_hbm, in_hbm, out_hbm,
              acc_sh, src_v, dst_v, val_v, rows_v, zbuf, sem):
    core = lax.axis_index("c")
    sid = lax.axis_index("s")
    base = core * HALF

    def zbody(r, _):
        for c in range(NCLASS // LANES):
            zbuf[r, pl.ds(c * LANES, LANES)] = jnp.zeros((LANES,), jnp.float32)
        return 0
    lax.fori_loop(0, ZROWS, zbody, 0)
    for q in range(ROWS_PER_TILE // ZROWS):
        pltpu.sync_copy(
            zbuf, acc_sh.at[pl.ds(sid * ROWS_PER_TILE + q * ZROWS, ZROWS)])
    plsc.subcore_barrier()

    def chunk_body(ci, _):
        row0 = sid * (EDGES_PER_TILE // EC) + ci * CHUNK_ROWS
        pltpu.sync_copy(src_hbm.at[pl.ds(row0, CHUNK_ROWS)], src_v)
        pltpu.sync_copy(dst_hbm.at[pl.ds(row0, CHUNK_ROWS)], dst_v)
        pltpu.sync_copy(val_hbm.at[pl.ds(row0, CHUNK_ROWS)], val_v)

        descs = [
            pltpu.async_copy(in_hbm.at[src_v.at[j]], rows_v.at[j], sem)
            for j in range(CHUNK_ROWS)
        ]
        for j in range(CHUNK_ROWS):
            for kk in range(EC // LANES):
                sl = pl.ds(kk * LANES, LANES)
                rel = dst_v[j, sl] - base
                inr = (rel >= 0) & (rel < HALF)
                dst_v[j, sl] = jnp.where(inr, rel, 0)
                val_v[j, sl] = jnp.where(inr, val_v[j, sl], 0.0)
        for d in descs:
            d.wait()

        for j in range(CHUNK_ROWS):
            def mbody(g, _, j=j):
                g16 = pl.multiple_of(g * LANES, LANES)
                vals16 = val_v[j, pl.ds(g16, LANES)]
                for l in range(LANES):
                    vb = jnp.full((LANES,), vals16[l], jnp.float32)
                    e = g16 + l
                    for c in range(NCLASS // LANES):
                        sl = pl.ds(c * LANES, LANES)
                        rows_v[j, e, sl] = rows_v[j, e, sl] * vb
                return 0
            lax.fori_loop(0, EC // LANES, mbody, 0)

        for j in range(CHUNK_ROWS):
            pltpu.sync_copy(rows_v.at[j], acc_sh.at[dst_v.at[j]], add=True)
        return 0

    lax.fori_loop(0, NCHUNKS, chunk_body, 0)
    plsc.subcore_barrier()

    pltpu.sync_copy(
        acc_sh.at[pl.ds(sid * ROWS_PER_TILE, ROWS_PER_TILE)],
        out_hbm.at[pl.ds(base + sid * ROWS_PER_TILE, ROWS_PER_TILE)])


_hop = functools.partial(
    pl.kernel,
    out_type=jax.ShapeDtypeStruct((NPAD, NCLASS), jnp.float32),
    mesh=plsc.VectorSubcoreMesh(core_axis_name="c", subcore_axis_name="s"),
    compiler_params=pltpu.CompilerParams(use_tc_tiling_on_sc=False),
    scratch_types=[
        pltpu.VMEM_SHARED((HALF, NCLASS), jnp.float32),
        pltpu.VMEM((CHUNK_ROWS, EC), jnp.int32),
        pltpu.VMEM((CHUNK_ROWS, EC), jnp.int32),
        pltpu.VMEM((CHUNK_ROWS, EC), jnp.float32),
        pltpu.VMEM((CHUNK_ROWS, EC, NCLASS), jnp.float32),
        pltpu.VMEM((ZROWS, NCLASS), jnp.float32),
        pltpu.SemaphoreType.DMA,
    ],
)(_hop_body)


def kernel(x, adj_values, W1, b1, W2, b2, adj_indices):
    dst = adj_indices[0]
    src = adj_indices[1]
    epad = EPAD - E
    srcp = jnp.concatenate([src, jnp.zeros((epad,), jnp.int32)]).reshape(EROWS, EC)
    dstp = jnp.concatenate([dst, jnp.zeros((epad,), jnp.int32)]).reshape(EROWS, EC)
    valp = jnp.concatenate(
        [adj_values, jnp.zeros((epad,), jnp.float32)]).reshape(EROWS, EC)
    xp = jnp.pad(x, ((0, NPAD - N), (0, 0)))

    h = _mlp(xp, W1, b1, W2, b2)
    for _ in range(K):
        h = _hop(srcp, dstp, valp, h)
    return h[:N]


# speedup vs baseline: 2.5668x; 1.5817x over previous
import functools

import jax
import jax.numpy as jnp
from jax import lax
from jax.experimental import pallas as pl
from jax.experimental.pallas import tpu as pltpu
from jax.experimental.pallas import tpu_sc as plsc

N = 50000
E = 800000
NFEAT = 128
NHID = 128
NCLASS = 64
K = 8

NC = 2
NS = 16
LANES = 16

ROWS_PER_TILE = 1600
NPAD = NC * NS * ROWS_PER_TILE
HALF = NS * ROWS_PER_TILE

EC = 128
NCHUNKS = 399
NTRIPLES = NCHUNKS // 3
EDGES_PER_TILE = NCHUNKS * EC
EPAD = NS * EDGES_PER_TILE
EROWS = EPAD // EC
NBUF = 3


def _mlp_body(x_ref, w1_ref, b1_ref, w2_ref, b2_ref, o_ref):
    h = jnp.dot(x_ref[...], w1_ref[...], preferred_element_type=jnp.float32)
    h = jnp.maximum(h + b1_ref[...], 0.0)
    o = jnp.dot(h, w2_ref[...], preferred_element_type=jnp.float32)
    o_ref[...] = o + b2_ref[...]


_MLP_BM = 2048
_MLP_GRID = NPAD // _MLP_BM


def _mlp(xp, W1, b1, W2, b2):
    return pl.pallas_call(
        _mlp_body,
        grid=(_MLP_GRID,),
        in_specs=[
            pl.BlockSpec((_MLP_BM, NFEAT), lambda i: (i, 0)),
            pl.BlockSpec((NFEAT, NHID), lambda i: (0, 0)),
            pl.BlockSpec((1, NHID), lambda i: (0, 0)),
            pl.BlockSpec((NHID, NCLASS), lambda i: (0, 0)),
            pl.BlockSpec((1, NCLASS), lambda i: (0, 0)),
        ],
        out_specs=pl.BlockSpec((_MLP_BM, NCLASS), lambda i: (i, 0)),
        out_shape=jax.ShapeDtypeStruct((NPAD, NCLASS), jnp.float32),
    )(xp, W1, b1.reshape(1, NHID), W2, b2.reshape(1, NCLASS))


def _hop_body(src_hbm, dst_hbm, val_hbm, in_hbm, zeros_hbm, out_hbm,
              acc_sh, srcb, dstb, valb, rowsb,
              esem0, esem1, esem2, gsem0, gsem1, gsem2, ssem0, ssem1, ssem2):
    esem = (esem0, esem1, esem2)
    gsem = (gsem0, gsem1, gsem2)
    ssem = (ssem0, ssem1, ssem2)
    core = lax.axis_index("c")
    sid = lax.axis_index("s")
    base = core * HALF
    row0 = sid * NCHUNKS

    pltpu.sync_copy(zeros_hbm, acc_sh.at[pl.ds(sid * ROWS_PER_TILE, ROWS_PER_TILE)])
    plsc.subcore_barrier()

    edge_bufs = ((src_hbm, srcb), (dst_hbm, dstb), (val_hbm, valb))

    def fire_edges(x, r):
        for h, b in edge_bufs:
            pltpu.async_copy(h.at[r], b.at[x], esem[x])

    def drain_edges(x):
        for h, b in edge_bufs:
            pltpu.make_async_copy(h.at[row0], b.at[x], esem[x]).wait()

    def fire_gather(x):
        pltpu.async_copy(in_hbm.at[srcb.at[x]], rowsb.at[x], gsem[x])

    def drain_gather(x):
        pltpu.make_async_copy(in_hbm.at[srcb.at[x]], rowsb.at[x], gsem[x]).wait()

    def fire_scatter(x):
        pltpu.async_copy(rowsb.at[x], acc_sh.at[dstb.at[x]], ssem[x], add=True)

    def drain_scatter(x):
        pltpu.make_async_copy(rowsb.at[x], acc_sh.at[dstb.at[x]], ssem[x]).wait()

    def mask_buf(x):
        for g in range(EC // LANES):
            sl = pl.ds(g * LANES, LANES)
            rel = dstb[x, sl] - base
            inr = (rel >= 0) & (rel < HALF)
            dstb[x, sl] = jnp.where(inr, rel, 0)
            valb[x, sl] = jnp.where(inr, valb[x, sl], 0.0)

    def mult_buf(x):
        def mbody(g, _):
            g16 = pl.multiple_of(g * LANES, LANES)
            v16 = valb[x, pl.ds(g16, LANES)]
            for l in range(LANES):
                vb = jnp.full((LANES,), v16[l], jnp.float32)
                e = g16 + l
                for c in range(NCLASS // LANES):
                    sl = pl.ds(c * LANES, LANES)
                    rowsb[x, e, sl] = rowsb[x, e, sl] * vb
            return 0
        lax.fori_loop(0, EC // LANES, mbody, 0)

    fire_edges(0, row0)
    fire_edges(1, row0 + 1)
    drain_edges(0)
    mask_buf(0)
    fire_gather(0)

    def triple(i, _):
        for s in range(NBUF):
            k = i * NBUF + s
            nxt = (s + 1) % NBUF
            prv = (s + 2) % NBUF

            @pl.when(k < NCHUNKS - 1)
            def _():
                drain_edges(nxt)
                mask_buf(nxt)
                fire_gather(nxt)

            drain_gather(s)
            mult_buf(s)
            fire_scatter(s)

            @pl.when(k >= 1)
            def _():
                drain_scatter(prv)

            @pl.when(k < NCHUNKS - 2)
            def _():
                fire_edges(prv, row0 + k + 2)
        return 0

    lax.fori_loop(0, NTRIPLES, triple, 0)
    drain_scatter((NCHUNKS - 1) % NBUF)
    plsc.subcore_barrier()

    pltpu.sync_copy(
        acc_sh.at[pl.ds(sid * ROWS_PER_TILE, ROWS_PER_TILE)],
        out_hbm.at[pl.ds(base + sid * ROWS_PER_TILE, ROWS_PER_TILE)])


_hop = functools.partial(
    pl.kernel,
    out_type=jax.ShapeDtypeStruct((NPAD, NCLASS), jnp.float32),
    mesh=plsc.VectorSubcoreMesh(core_axis_name="c", subcore_axis_name="s"),
    compiler_params=pltpu.CompilerParams(use_tc_tiling_on_sc=False),
    scratch_types=[
        pltpu.VMEM_SHARED((HALF, NCLASS), jnp.float32),
        pltpu.VMEM((NBUF, EC), jnp.int32),
        pltpu.VMEM((NBUF, EC), jnp.int32),
        pltpu.VMEM((NBUF, EC), jnp.float32),
        pltpu.VMEM((NBUF, EC, NCLASS), jnp.float32),
    ] + [pltpu.SemaphoreType.DMA] * 9,
)(_hop_body)


def kernel(x, adj_values, W1, b1, W2, b2, adj_indices):
    dst = adj_indices[0]
    src = adj_indices[1]
    epad = EPAD - E
    srcp = jnp.concatenate([src, jnp.zeros((epad,), jnp.int32)]).reshape(EROWS, EC)
    dstp = jnp.concatenate([dst, jnp.zeros((epad,), jnp.int32)]).reshape(EROWS, EC)
    valp = jnp.concatenate(
        [adj_values, jnp.zeros((epad,), jnp.float32)]).reshape(EROWS, EC)
    xp = jnp.pad(x, ((0, NPAD - N), (0, 0)))
    zeros_blob = jnp.zeros((ROWS_PER_TILE, NCLASS), jnp.float32)

    h = _mlp(xp, W1, b1, W2, b2)
    for _ in range(K):
        h = _hop(srcp, dstp, valp, h, zeros_blob)
    return h[:N]
